# baseline (speedup 1.0000x reference)
import functools

import jax
import jax.numpy as jnp
from jax import lax
from jax.experimental import pallas as pl
from jax.experimental.pallas import tpu as pltpu
from jax.experimental.pallas import tpu_sc as plsc


def kernel(x, y, name_embedding, value_table, mask_embedding):
    B, L = x.shape
    V, D = value_table.shape
    DP = 2 * D
    LP = 104
    NW = 32
    BPW = B // NW
    CB = 8
    NCH = BPW // CB
    R = CB * L
    ND = D // 16

    mesh = plsc.VectorSubcoreMesh(core_axis_name="c", subcore_axis_name="s")

    @functools.partial(
        pl.kernel,
        mesh=mesh,
        compiler_params=pltpu.CompilerParams(use_tc_tiling_on_sc=True,
                                             needs_layout_passes=False),
        out_type=jax.ShapeDtypeStruct((B * LP, DP), jnp.float32),
        scratch_types=[
            pltpu.VMEM((CB, LP), jnp.int32),
            pltpu.VMEM((CB * LP, DP), jnp.float32),
            pltpu.VMEM((L, D), jnp.float32),
            pltpu.VMEM((D,), jnp.float32),
            pltpu.VMEM((BPW + 16,), jnp.int32),
            pltpu.SemaphoreType.DMA,
        ],
    )
    def run(x_hbm, y_hbm, name_hbm, table_hbm, mask_hbm, out_hbm,
            idx_v, rows_v, name_v, mask_v, y_vmem, gsem):
        wid = lax.axis_index("s") * 2 + lax.axis_index("c")
        bbase = wid * BPW
        pltpu.sync_copy(name_hbm, name_v)
        pltpu.sync_copy(mask_hbm, mask_v)
        pltpu.sync_copy(y_hbm.at[pl.ds(bbase, BPW)], y_vmem.at[pl.ds(0, BPW)])
        lane = lax.iota(jnp.int32, 16)

        def chunk_body(c, carry):
            b0 = bbase + c * CB
            pltpu.sync_copy(x_hbm.at[pl.ds(b0, CB)], idx_v)
            copies = [
                pltpu.async_copy(table_hbm.at[idx_v.at[j]],
                                 rows_v.at[pl.ds(j * LP, LP)], gsem)
                for j in range(CB)
            ]
            for cp in copies:
                cp.wait()

            def add_l(l, carry2):
                nm = [name_v[l, pl.ds(16 * d, 16)] for d in range(ND)]
                for b in range(CB):
                    r = b * LP + l
                    for d in range(ND):
                        rows_v[r, pl.ds(16 * d, 16)] = (
                            rows_v[r, pl.ds(16 * d, 16)] + nm[d])
                return carry2

            lax.fori_loop(0, L, add_l, 0)

            y16 = y_vmem[pl.ds(c * CB, 16)]
            for b in range(CB):
                yb = jnp.max(jnp.where(lane == b, y16, 0))
                r = b * LP + yb
                for d in range(ND):
                    rows_v[r, pl.ds(16 * d, 16)] = (
                        mask_v[pl.ds(16 * d, 16)]
                        + name_v[yb, pl.ds(16 * d, 16)])

            pltpu.sync_copy(rows_v, out_hbm.at[pl.ds(b0 * LP, CB * LP)])
            return carry

        lax.fori_loop(0, NCH, chunk_body, 0)

    vt_pad = jnp.pad(value_table, ((0, 0), (0, DP - D)))
    x_pad = jnp.pad(x, ((0, 0), (0, LP - L)))
    out = run(x_pad, y, name_embedding, vt_pad, mask_embedding)
    return out.reshape(B, LP, DP)[:, :L, :D]


# speedup vs baseline: 3.5134x; 3.5134x over previous
import functools

import jax
import jax.numpy as jnp
from jax import lax
from jax.experimental import pallas as pl
from jax.experimental.pallas import tpu as pltpu
from jax.experimental.pallas import tpu_sc as plsc


def kernel(x, y, name_embedding, value_table, mask_embedding):
    B, L = x.shape
    V, D = value_table.shape
    LP = 104
    DP = 2 * D
    NW = 32
    BPW = B // NW
    CB = 16
    NCH = BPW // CB
    R = CB * L
    ND = D // 16

    mesh = plsc.VectorSubcoreMesh(core_axis_name="c", subcore_axis_name="s")

    @functools.partial(
        pl.kernel,
        mesh=mesh,
        compiler_params=pltpu.CompilerParams(use_tc_tiling_on_sc=False,
                                             needs_layout_passes=False),
        out_type=jax.ShapeDtypeStruct((B, LP, DP), jnp.float32),
        scratch_types=[
            pltpu.VMEM((CB, L), jnp.int32),
            pltpu.VMEM((R, D), jnp.float32),
            pltpu.VMEM((L, D), jnp.float32),
            pltpu.VMEM((D,), jnp.float32),
            pltpu.VMEM((BPW,), jnp.int32),
            pltpu.SemaphoreType.DMA,
        ],
    )
    def run(x_hbm, y_hbm, name_hbm, table_hbm, mask_hbm, out_hbm,
            idx_v, rows_v, name_v, mask_v, y_vmem, gsem):
        wid = lax.axis_index("s") * 2 + lax.axis_index("c")
        bbase = wid * BPW
        pltpu.sync_copy(name_hbm, name_v)
        pltpu.sync_copy(mask_hbm, mask_v)
        pltpu.sync_copy(y_hbm.at[pl.ds(bbase, BPW)], y_vmem)
        lane = lax.iota(jnp.int32, 16)

        def chunk_body(c, carry):
            b0 = bbase + c * CB
            pltpu.sync_copy(x_hbm.at[pl.ds(b0, CB)], idx_v)
            copies = [
                pltpu.async_copy(table_hbm.at[idx_v.at[j]],
                                 rows_v.at[pl.ds(j * L, L)], gsem)
                for j in range(CB)
            ]
            for cp in copies:
                cp.wait()

            def add_l(l, carry2):
                nm = [name_v[l, pl.ds(16 * d, 16)] for d in range(ND)]
                for b in range(CB):
                    r = b * L + l
                    for d in range(ND):
                        rows_v[r, pl.ds(16 * d, 16)] = (
                            rows_v[r, pl.ds(16 * d, 16)] + nm[d])
                return carry2

            lax.fori_loop(0, L, add_l, 0)

            y16 = y_vmem[pl.ds(c * CB, 16)]
            for b in range(CB):
                yb = jnp.max(jnp.where(lane == b, y16, 0))
                r = b * L + yb
                for d in range(ND):
                    rows_v[r, pl.ds(16 * d, 16)] = (
                        mask_v[pl.ds(16 * d, 16)]
                        + name_v[yb, pl.ds(16 * d, 16)])

            for j in range(CB):
                pltpu.sync_copy(
                    rows_v.at[pl.ds(j * L, L)],
                    out_hbm.at[b0 + j, pl.ds(0, L), pl.ds(0, D)])
            return carry

        lax.fori_loop(0, NCH, chunk_body, 0)

    out = run(x, y, name_embedding, value_table, mask_embedding)
    return out[:, :L, :D]
